# Initial kernel scaffold; baseline (speedup 1.0000x reference)
import jax
import jax.numpy as jnp
from jax.experimental import pallas as pl


def kernel(h, mask, W, att_src, att_dst, bias, bn1_w, bn1_b, W1, b1, W2, b2, bn2_w, bn2_b):
    raise NotImplementedError("write your pallas kernel here")


# speedup vs baseline: 18.4678x; 18.4678x over previous
import functools

import jax
import jax.numpy as jnp
from jax import lax
from jax.experimental import pallas as pl
from jax.experimental.pallas import tpu as pltpu
from jax.experimental.pallas import tpu_sc as plsc

N = 10000
D = 128
H = 8
C = 128
FFD = 512
XW = H * C
CW = 64
XH = H * CW

NC = 2
NS = 16
NW = NC * NS

NR = 10240
RPT = NR // NS
NPAD = N

ET = 160000 + N
EPT = 5376
EPAD = EPT * NW
K1 = 128
N1 = EPT // K1
K2 = 32
N2 = EPT // K2

BR = 640
BN = 1000

_sc_params = functools.partial(pltpu.CompilerParams, use_tc_tiling_on_sc=False)


def _mesh():
    return plsc.VectorSubcoreMesh(core_axis_name="c", subcore_axis_name="s",
                                  num_cores=NC, num_subcores=NS)


def _gather16(vec, idx):
    dn = lax.GatherDimensionNumbers(offset_dims=(), collapsed_slice_dims=(0,),
                                    start_index_map=(0,))
    return lax.gather(vec, idx[:, None], dn, (1,),
                      mode=lax.GatherScatterMode.PROMISE_IN_BOUNDS)


def _splat(vec, h):
    return _gather16(vec, jnp.full((16,), h, jnp.int32))


def _tc1_body(h_ref, wa_ref, wb_ref, s_ref, d_ref, gs_ref, gd_ref,
              xa_ref, xb_ref, t_ref):
    xa = jnp.dot(h_ref[...], wa_ref[...], preferred_element_type=jnp.float32)
    xb = jnp.dot(h_ref[...], wb_ref[...], preferred_element_type=jnp.float32)
    xa_ref[...] = xa
    xb_ref[...] = xb
    x = jnp.concatenate([xa, xb], axis=1)
    t_ref[...] = jnp.dot(x * s_ref[...], gs_ref[...],
                         preferred_element_type=jnp.float32) \
        + jnp.dot(x * d_ref[...], gd_ref[...],
                  preferred_element_type=jnp.float32)


def _tc1(hpad, Wa, Wb, sflat, dflat, gs, gd):
    return pl.pallas_call(
        _tc1_body,
        grid=(NR // BR,),
        in_specs=[
            pl.BlockSpec((BR, D), lambda i: (i, 0)),
            pl.BlockSpec((D, XH), lambda i: (0, 0)),
            pl.BlockSpec((D, XH), lambda i: (0, 0)),
            pl.BlockSpec((1, XW), lambda i: (0, 0)),
            pl.BlockSpec((1, XW), lambda i: (0, 0)),
            pl.BlockSpec((XW, 128), lambda i: (0, 0)),
            pl.BlockSpec((XW, 128), lambda i: (0, 0)),
        ],
        out_specs=[
            pl.BlockSpec((BR, XH), lambda i: (i, 0)),
            pl.BlockSpec((BR, XH), lambda i: (i, 0)),
            pl.BlockSpec((BR, 128), lambda i: (i, 0)),
        ],
        out_shape=[
            jax.ShapeDtypeStruct((NR, XH), jnp.float32),
            jax.ShapeDtypeStruct((NR, XH), jnp.float32),
            jax.ShapeDtypeStruct((NR, 128), jnp.float32),
        ],
    )(hpad, Wa, Wb, sflat, dflat, gs, gd)


def _sc1_body(src_hbm, dsta_hbm, t_hbm,
              w_hbm, den_hbm,
              srcv, dstav, ps, pd, wv, zb, dsh,
              sA0, sA1, sB0, sB1, sw0, sw1, sc0, sc1_):
    cid = lax.axis_index("c")
    sid = lax.axis_index("s")
    wid = cid * NS + sid
    sA = (sA0, sA1)
    sB = (sB0, sB1)
    sw = (sw0, sw1)
    ssc = (sc0, sc1_)

    @pl.loop(0, RPT)
    def _(r):
        zb[r, :] = jnp.zeros((16,), jnp.float32)

    pltpu.sync_copy(zb, dsh.at[pl.ds(sid * RPT, RPT)])
    plsc.subcore_barrier()

    ebase = wid * EPT
    pltpu.sync_copy(src_hbm.at[pl.ds(ebase, EPT)], srcv)
    pltpu.sync_copy(dsta_hbm.at[wid], dstav)

    def start_gather(i, b):
        pltpu.async_copy(t_hbm.at[srcv.at[pl.ds(i * K1, K1)]],
                         ps.at[b], sA[b])
        pltpu.async_copy(t_hbm.at[dstav.at[i]], pd.at[b], sB[b])

    start_gather(0, 0)

    @pl.loop(0, N1, step=2)
    def _(i0):
        for b in range(2):
            i = i0 + b

            @pl.when(i + 1 < N1)
            def _():
                start_gather(i + 1, 1 - b)

            pltpu.make_async_copy(t_hbm.at[srcv.at[pl.ds(0, K1)]],
                                  ps.at[b], sA[b]).wait()
            pltpu.make_async_copy(t_hbm.at[dstav.at[0]],
                                  pd.at[b], sB[b]).wait()

            @pl.when(i >= 2)
            def _():
                pltpu.make_async_copy(wv.at[b], w_hbm.at[0], sw[b]).wait()
                pltpu.make_async_copy(wv.at[b], dsh.at[dstav.at[0]],
                                      ssc[b]).wait()

            sh8 = (lax.broadcasted_iota(jnp.int32, (16,), 0) & 7) + 8

            @pl.loop(0, K1)
            def _(j):
                pdr = _gather16(pd[b, j, pl.ds(0, 16)], sh8)
                a = ps[b, j, pl.ds(0, 16)] + pdr
                a = jnp.where(a >= 0.0, a, a * jnp.float32(0.2))
                wv[b, j, :] = jnp.exp(a)

            pltpu.async_copy(wv.at[b], w_hbm.at[wid * N1 + i], sw[b])
            pltpu.async_copy(wv.at[b], dsh.at[dstav.at[i]], ssc[b],
                             add=True)

    for b in range(2):
        pltpu.make_async_copy(wv.at[b], w_hbm.at[0], sw[b]).wait()
        pltpu.make_async_copy(wv.at[b], dsh.at[dstav.at[0]], ssc[b]).wait()

    plsc.subcore_barrier()
    pltpu.sync_copy(dsh.at[pl.ds(sid * RPT, RPT)],
                    den_hbm.at[cid, pl.ds(sid * RPT, RPT)])


def _sc1(src_flat, dst_a, t_tab):
    f = pl.kernel(
        _sc1_body,
        out_type=[
            jax.ShapeDtypeStruct((EPAD // K1, K1, 16), jnp.float32),
            jax.ShapeDtypeStruct((NC, NR, 16), jnp.float32),
        ],
        mesh=_mesh(),
        compiler_params=_sc_params(),
        scratch_types=[
            pltpu.VMEM((EPT,), jnp.int32),
            pltpu.VMEM((N1, K1), jnp.int32),
            pltpu.VMEM((2, K1, 128), jnp.float32),
            pltpu.VMEM((2, K1, 128), jnp.float32),
            pltpu.VMEM((2, K1, 16), jnp.float32),
            pltpu.VMEM((RPT, 16), jnp.float32),
            pltpu.VMEM_SHARED((NR, 16), jnp.float32),
            pltpu.SemaphoreType.DMA,
            pltpu.SemaphoreType.DMA,
            pltpu.SemaphoreType.DMA,
            pltpu.SemaphoreType.DMA,
            pltpu.SemaphoreType.DMA,
            pltpu.SemaphoreType.DMA,
            pltpu.SemaphoreType.DMA,
            pltpu.SemaphoreType.DMA,
        ],
    )
    return f(src_flat, dst_a, t_tab)


def _tc2_body(den_ref, invd_ref):
    d = den_ref[0] + den_ref[1]
    v = 1.0 / ((d + jnp.float32(1e-16)) * jnp.float32(H))
    invd_ref[...] = jnp.concatenate(
        [v, jnp.zeros((NR, 112), jnp.float32)], axis=1)


def _tc2(den_p):
    return pl.pallas_call(
        _tc2_body,
        out_shape=jax.ShapeDtypeStruct((NR, 128), jnp.float32),
    )(den_p)


def _sc15_body(dsta_hbm, w_hbm, invd_hbm,
               co_hbm,
               dstav, ib, wv, cv,
               si0, si1, sw0, sw1, so0, so1):
    cid = lax.axis_index("c")
    sid = lax.axis_index("s")
    wid = cid * NS + sid
    si = (si0, si1)
    sw = (sw0, sw1)
    so = (so0, so1)

    pltpu.sync_copy(dsta_hbm.at[wid], dstav)

    def start_loads(i, b):
        pltpu.async_copy(invd_hbm.at[dstav.at[i]], ib.at[b], si[b])
        pltpu.async_copy(w_hbm.at[wid * N1 + i], wv.at[b], sw[b])

    start_loads(0, 0)

    @pl.loop(0, N1, step=2)
    def _(i0):
        for b in range(2):
            i = i0 + b

            @pl.when(i + 1 < N1)
            def _():
                start_loads(i + 1, 1 - b)

            pltpu.make_async_copy(invd_hbm.at[dstav.at[0]],
                                  ib.at[b], si[b]).wait()
            pltpu.make_async_copy(w_hbm.at[0], wv.at[b], sw[b]).wait()

            @pl.when(i >= 2)
            def _():
                pltpu.make_async_copy(cv.at[b], co_hbm.at[0], so[b]).wait()

            @pl.loop(0, K1)
            def _(j):
                cv[b, j, :] = wv[b, j, :] * ib[b, j, pl.ds(0, 16)]

            pltpu.async_copy(cv.at[b], co_hbm.at[wid * N1 + i], so[b])

    for b in range(2):
        pltpu.make_async_copy(cv.at[b], co_hbm.at[0], so[b]).wait()


def _sc15(dst_a, w1_, invd):
    f = pl.kernel(
        _sc15_body,
        out_type=jax.ShapeDtypeStruct((EPAD // K1, K1, 16), jnp.float32),
        mesh=_mesh(),
        compiler_params=_sc_params(),
        scratch_types=[
            pltpu.VMEM((N1, K1), jnp.int32),
            pltpu.VMEM((2, K1, 128), jnp.float32),
            pltpu.VMEM((2, K1, 16), jnp.float32),
            pltpu.VMEM((2, K1, 16), jnp.float32),
            pltpu.SemaphoreType.DMA,
            pltpu.SemaphoreType.DMA,
            pltpu.SemaphoreType.DMA,
            pltpu.SemaphoreType.DMA,
            pltpu.SemaphoreType.DMA,
            pltpu.SemaphoreType.DMA,
        ],
    )
    return f(dst_a, w1_, invd)


def _sc2_body(src_hbm, dstc_hbm, w_hbm, x_hbm,
              out_hbm,
              srcv, dstcv, xb, wb, ob, zb, osh,
              sx0, sx1, sw0, sw1, sc0, sc1_):
    cid = lax.axis_index("c")
    sid = lax.axis_index("s")
    wid = cid * NS + sid
    sx = (sx0, sx1)
    sw = (sw0, sw1)
    ssc = (sc0, sc1_)

    @pl.loop(0, 16)
    def _(r):
        for ci in range(CW // 16):
            zb[r, pl.ds(ci * 16, 16)] = jnp.zeros((16,), jnp.float32)

    @pl.loop(0, RPT // 16)
    def _(t):
        pltpu.sync_copy(zb, osh.at[pl.ds(sid * RPT + t * 16, 16)])

    plsc.subcore_barrier()

    ebase = wid * EPT
    pltpu.sync_copy(src_hbm.at[pl.ds(ebase, EPT)], srcv)
    pltpu.sync_copy(dstc_hbm.at[wid], dstcv)

    def start_loads(i, b):
        pltpu.async_copy(x_hbm.at[srcv.at[pl.ds(i * K2, K2)]], xb.at[b],
                         sx[b])
        pltpu.async_copy(w_hbm.at[wid * N2 + i], wb.at[b], sw[b])

    start_loads(0, 0)

    @pl.loop(0, N2, step=2)
    def _(i0):
        for b in range(2):
            i = i0 + b

            @pl.when(i + 1 < N2)
            def _():
                start_loads(i + 1, 1 - b)

            pltpu.make_async_copy(x_hbm.at[srcv.at[pl.ds(0, K2)]],
                                  xb.at[b], sx[b]).wait()
            pltpu.make_async_copy(w_hbm.at[0], wb.at[b], sw[b]).wait()

            @pl.when(i >= 2)
            def _():
                pltpu.make_async_copy(ob.at[b], osh.at[dstcv.at[0]],
                                      ssc[b]).wait()

            @pl.loop(0, K2)
            def _(j):
                crow = wb[b, j, :]
                acc = [jnp.zeros((16,), jnp.float32)
                       for _ in range(CW // 16)]
                for h in range(H):
                    s = _splat(crow, h)
                    for ci in range(CW // 16):
                        acc[ci] = acc[ci] + s * xb[b, j,
                                                   pl.ds(h * CW + ci * 16,
                                                         16)]
                for ci in range(CW // 16):
                    ob[b, j, pl.ds(ci * 16, 16)] = acc[ci]

            pltpu.async_copy(ob.at[b], osh.at[dstcv.at[i]], ssc[b],
                             add=True)

    for b in range(2):
        pltpu.make_async_copy(ob.at[b], osh.at[dstcv.at[0]], ssc[b]).wait()

    plsc.subcore_barrier()
    pltpu.sync_copy(osh.at[pl.ds(sid * RPT, RPT)],
                    out_hbm.at[cid, pl.ds(sid * RPT, RPT)])


def _sc2(src_flat, dst_c, co2, xhalf):
    f = pl.kernel(
        _sc2_body,
        out_type=jax.ShapeDtypeStruct((NC, NR, CW), jnp.float32),
        mesh=_mesh(),
        compiler_params=_sc_params(),
        scratch_types=[
            pltpu.VMEM((EPT,), jnp.int32),
            pltpu.VMEM((N2, K2), jnp.int32),
            pltpu.VMEM((2, K2, XH), jnp.float32),
            pltpu.VMEM((2, K2, 16), jnp.float32),
            pltpu.VMEM((2, K2, CW), jnp.float32),
            pltpu.VMEM((16, CW), jnp.float32),
            pltpu.VMEM_SHARED((NR, CW), jnp.float32),
            pltpu.SemaphoreType.DMA,
            pltpu.SemaphoreType.DMA,
            pltpu.SemaphoreType.DMA,
            pltpu.SemaphoreType.DMA,
            pltpu.SemaphoreType.DMA,
            pltpu.SemaphoreType.DMA,
        ],
    )
    return f(src_flat, dst_c, co2, xhalf)


def _tc3a_body(opa_ref, opb_ref, h_ref, bias_ref, h1_ref, st_ref):
    i = pl.program_id(0)
    ga = opa_ref[0] + opa_ref[1]
    gb = opb_ref[0] + opb_ref[1]
    g = jnp.concatenate([ga, gb], axis=1)
    h1 = h_ref[...] + g + bias_ref[...]
    h1_ref[...] = h1
    s = jnp.sum(h1, axis=0)[None, :]
    sq = jnp.sum(h1 * h1, axis=0)[None, :]
    st = jnp.concatenate([s, sq], axis=0)

    @pl.when(i == 0)
    def _():
        st_ref[...] = jnp.zeros_like(st_ref)

    st_ref[...] += st


def _tc3a(out_pa, out_pb, h, bias2d):
    return pl.pallas_call(
        _tc3a_body,
        grid=(N // BN,),
        in_specs=[
            pl.BlockSpec((NC, BN, CW), lambda i: (0, i, 0)),
            pl.BlockSpec((NC, BN, CW), lambda i: (0, i, 0)),
            pl.BlockSpec((BN, D), lambda i: (i, 0)),
            pl.BlockSpec((1, D), lambda i: (0, 0)),
        ],
        out_specs=[
            pl.BlockSpec((BN, D), lambda i: (i, 0)),
            pl.BlockSpec((2, D), lambda i: (0, 0)),
        ],
        out_shape=[
            jax.ShapeDtypeStruct((N, D), jnp.float32),
            jax.ShapeDtypeStruct((2, D), jnp.float32),
        ],
    )(out_pa, out_pb, h, bias2d)


def _tc3b_body(h1_ref, st_ref, g1_ref, b1_ref, w1_ref, bb1_ref, w2_ref,
               bb2_ref, h2_ref, st2_ref):
    i = pl.program_id(0)
    mu = st_ref[0, :] / jnp.float32(N)
    var = st_ref[1, :] / jnp.float32(N) - mu * mu
    rstd = lax.rsqrt(var + jnp.float32(1e-5))
    h1n = (h1_ref[...] - mu[None, :]) * (rstd * g1_ref[0, :])[None, :] \
        + b1_ref[...]
    t = jnp.dot(h1n, w1_ref[...], preferred_element_type=jnp.float32)
    t = jnp.maximum(t + bb1_ref[...], 0.0)
    ff = jnp.dot(t, w2_ref[...], preferred_element_type=jnp.float32) \
        + bb2_ref[...]
    h2 = h1n + ff
    h2_ref[...] = h2
    s = jnp.sum(h2, axis=0)[None, :]
    sq = jnp.sum(h2 * h2, axis=0)[None, :]
    st = jnp.concatenate([s, sq], axis=0)

    @pl.when(i == 0)
    def _():
        st2_ref[...] = jnp.zeros_like(st2_ref)

    st2_ref[...] += st


def _tc3b(h1, st1, g1, b1, W1, bb1, W2, bb2):
    return pl.pallas_call(
        _tc3b_body,
        grid=(N // BN,),
        in_specs=[
            pl.BlockSpec((BN, D), lambda i: (i, 0)),
            pl.BlockSpec((2, D), lambda i: (0, 0)),
            pl.BlockSpec((1, D), lambda i: (0, 0)),
            pl.BlockSpec((1, D), lambda i: (0, 0)),
            pl.BlockSpec((D, FFD), lambda i: (0, 0)),
            pl.BlockSpec((1, FFD), lambda i: (0, 0)),
            pl.BlockSpec((FFD, D), lambda i: (0, 0)),
            pl.BlockSpec((1, D), lambda i: (0, 0)),
        ],
        out_specs=[
            pl.BlockSpec((BN, D), lambda i: (i, 0)),
            pl.BlockSpec((2, D), lambda i: (0, 0)),
        ],
        out_shape=[
            jax.ShapeDtypeStruct((N, D), jnp.float32),
            jax.ShapeDtypeStruct((2, D), jnp.float32),
        ],
    )(h1, st1, g1, b1, W1, bb1, W2, bb2)


def _tc3c_body(h2_ref, st_ref, g_ref, b_ref, o_ref):
    mu = st_ref[0, :] / jnp.float32(N)
    var = st_ref[1, :] / jnp.float32(N) - mu * mu
    rstd = lax.rsqrt(var + jnp.float32(1e-5))
    o_ref[...] = (h2_ref[...] - mu[None, :]) * (rstd * g_ref[0, :])[None, :] \
        + b_ref[...]


def _tc3c(h2u, st2, g2, b2):
    return pl.pallas_call(
        _tc3c_body,
        grid=(N // BN,),
        in_specs=[
            pl.BlockSpec((BN, D), lambda i: (i, 0)),
            pl.BlockSpec((2, D), lambda i: (0, 0)),
            pl.BlockSpec((1, D), lambda i: (0, 0)),
            pl.BlockSpec((1, D), lambda i: (0, 0)),
        ],
        out_specs=pl.BlockSpec((BN, D), lambda i: (i, 0)),
        out_shape=jax.ShapeDtypeStruct((N, D), jnp.float32),
    )(h2u, st2, g2, b2)


def kernel(h, mask, W, att_src, att_dst, bias, bn1_w, bn1_b, W1, b1,
           W2, b2, bn2_w, bn2_b):
    f32 = jnp.float32
    h = h.astype(f32)

    loop = jnp.arange(N, dtype=jnp.int32)
    src = jnp.concatenate([mask[0].astype(jnp.int32), loop])
    dst = jnp.concatenate([mask[1].astype(jnp.int32), loop])
    pad = jnp.full((EPAD - ET,), NPAD, jnp.int32)
    src = jnp.concatenate([src, pad])
    dst = jnp.concatenate([dst, pad])
    dst_a = dst.reshape(NW, N1, K1)
    dst_c = dst.reshape(NW, N2, K2)

    hpad = jnp.pad(h, ((0, NR - N), (0, 0)))
    W = W.astype(f32)

    W3 = W.reshape(D, H, C)
    Wa = W3[:, :, :CW].reshape(D, XH)
    Wb = W3[:, :, CW:].reshape(D, XH)

    s3 = att_src.astype(f32)
    d3 = att_dst.astype(f32)
    sflat = jnp.concatenate([s3[:, :CW].reshape(-1), s3[:, CW:].reshape(-1)],
                            0).reshape(1, XW)
    dflat = jnp.concatenate([d3[:, :CW].reshape(-1), d3[:, CW:].reshape(-1)],
                            0).reshape(1, XW)

    gk = jnp.kron(jnp.eye(H, dtype=f32), jnp.ones((CW, 1), f32))
    gk2 = jnp.concatenate([gk, gk], axis=0)
    gs = jnp.pad(gk2, ((0, 0), (0, 128 - H)))
    gd = jnp.pad(gk2, ((0, 0), (H, 128 - 2 * H)))

    xa, xb2, t_tab = _tc1(hpad, Wa, Wb, sflat, dflat, gs, gd)

    w1_, den_p = _sc1(src, dst_a, t_tab)
    invd = _tc2(den_p)
    co1 = _sc15(dst_a, w1_, invd)
    co2 = co1.reshape(EPAD // K2, K2, 16)

    out_pa = _sc2(src, dst_c, co2, xa)
    out_pb = _sc2(src, dst_c, co2, xb2)

    h1, st1 = _tc3a(out_pa, out_pb, h, bias.astype(f32).reshape(1, D))
    h2u, st2 = _tc3b(h1, st1, bn1_w.astype(f32).reshape(1, D),
                     bn1_b.astype(f32).reshape(1, D), W1.astype(f32),
                     b1.astype(f32).reshape(1, FFD), W2.astype(f32),
                     b2.astype(f32).reshape(1, D))
    return _tc3c(h2u, st2, bn2_w.astype(f32).reshape(1, D),
                 bn2_b.astype(f32).reshape(1, D))
